# baseline (speedup 1.0000x reference)
import functools

import jax
import jax.numpy as jnp
from jax import lax
from jax.experimental import pallas as pl
from jax.experimental.pallas import tpu as pltpu
from jax.experimental.pallas import tpu_sc as plsc

N = 10000
E = 320000
D_IN = 128
HID = 128
HG = 64
HH = 32
OUT = 47
LAYERS = 4

NTILE = 16
NCORE = 2
NPAD = 10112
STRIPE = NPAD // NTILE
CH = 256
NCHUNK = 80
EPT = CH * NCHUNK
EPAD = NTILE * EPT
NBUF = 5
LEAD = 3
SPLIT = 60
CD = 16


@functools.cache
def _sc_mesh():
  return plsc.VectorSubcoreMesh(
      core_axis_name="c", subcore_axis_name="s", num_cores=NCORE,
      num_subcores=NTILE)


@functools.cache
def _make_sc_agg():
  @functools.partial(
      pl.kernel,
      out_type=pltpu.HBM((NCORE, NPAD, HH), jnp.float32),
      mesh=_sc_mesh(),
      scratch_types=[
          pltpu.VMEM((NCHUNK, CH), jnp.int32),
          pltpu.VMEM((NCHUNK, CH), jnp.int32),
          [pltpu.VMEM((CH, HH), jnp.float32)] * NBUF,
          [pltpu.SemaphoreType.DMA] * NBUF,
          [pltpu.SemaphoreType.DMA] * NBUF,
          pltpu.VMEM_SHARED((NPAD, HH), jnp.float32),
          pltpu.VMEM_SHARED((NPAD, HH), jnp.float32),
      ],
      compiler_params=pltpu.CompilerParams(use_tc_tiling_on_sc=False),
  )
  def agg(tab_lo, tab_hi, src2d, dst2d, out, src_v, dst_v, rows, gsems,
          ssems, acc, tab_s):
    c = lax.axis_index("c")
    s = lax.axis_index("s")
    zv = jnp.zeros((16,), jnp.float32)

    def zfill(i, carry):
      r = i // (HH // 16)
      k = i % (HH // 16)
      rows[0][r, pl.ds(k * 16, 16)] = zv
      return carry

    lax.fori_loop(0, CH * HH // 16, zfill, 0)
    nz = STRIPE // CH
    for z in range(nz):
      pltpu.sync_copy(rows[0], acc.at[pl.ds(s * STRIPE + z * CH, CH)])
    rem = STRIPE - nz * CH
    if rem:
      pltpu.sync_copy(
          rows[0].at[pl.ds(0, rem)],
          acc.at[pl.ds(s * STRIPE + nz * CH, rem)])

    @pl.when(c == 0)
    def _():
      pltpu.sync_copy(tab_lo.at[pl.ds(s * STRIPE, STRIPE)],
                      tab_s.at[pl.ds(s * STRIPE, STRIPE)])

    @pl.when(c == 1)
    def _():
      pltpu.sync_copy(tab_hi.at[pl.ds(s * STRIPE, STRIPE)],
                      tab_s.at[pl.ds(s * STRIPE, STRIPE)])

    pltpu.sync_copy(src2d.at[s], src_v)
    pltpu.sync_copy(dst2d.at[s], dst_v)
    plsc.subcore_barrier()

    def gather_start(i, buf, sem):
      @pl.when(i < SPLIT)
      def _():
        pltpu.async_copy(tab_s.at[src_v.at[i]], buf, sem)

      @pl.when(jnp.logical_and(i >= SPLIT, c == 0))
      def _():
        pltpu.async_copy(tab_lo.at[src_v.at[i]], buf, sem)

      @pl.when(jnp.logical_and(i >= SPLIT, c == 1))
      def _():
        pltpu.async_copy(tab_hi.at[src_v.at[i]], buf, sem)

    for i in range(LEAD):
      gather_start(jnp.int32(i), rows[i], gsems[i])

    def body(g, carry):
      for b in range(NBUF):
        i = g * NBUF + b
        pltpu.make_async_copy(tab_s.at[src_v.at[i]], rows[b],
                              gsems[b]).wait()
        pltpu.async_copy(rows[b], acc.at[dst_v.at[i]], ssems[b], add=True)
        j = i + LEAD
        bj = (b + LEAD) % NBUF

        @pl.when(j < NCHUNK)
        def _():
          @pl.when(j >= NBUF)
          def _():
            pltpu.make_async_copy(rows[bj], acc.at[dst_v.at[0]],
                                  ssems[bj]).wait()
          gather_start(j, rows[bj], gsems[bj])

      return carry

    lax.fori_loop(0, NCHUNK // NBUF, body, 0)
    for b in range(NBUF):
      pltpu.make_async_copy(rows[b], acc.at[dst_v.at[0]], ssems[b]).wait()
    plsc.subcore_barrier()
    pltpu.sync_copy(acc.at[pl.ds(s * STRIPE, STRIPE)],
                    out.at[c].at[pl.ds(s * STRIPE, STRIPE)])

  return agg


@functools.cache
def _make_sc_count():

  @functools.partial(
      pl.kernel,
      out_type=pltpu.HBM((NCORE, NPAD, CD), jnp.float32),
      mesh=_sc_mesh(),
      scratch_types=[
          pltpu.VMEM((NCHUNK // 2, CH), jnp.int32),
          pltpu.VMEM((CH, CD), jnp.float32),
          pltpu.VMEM_SHARED((NPAD, CD), jnp.float32),
          pltpu.SemaphoreType.DMA,
      ],
      compiler_params=pltpu.CompilerParams(use_tc_tiling_on_sc=False),
  )
  def count(dst2d, ones, zeros, out, dst_v, ones_v, acc, ssem):
    c = lax.axis_index("c")
    s = lax.axis_index("s")
    pltpu.sync_copy(zeros.at[pl.ds(s * STRIPE, STRIPE)],
                    acc.at[pl.ds(s * STRIPE, STRIPE)])
    pltpu.sync_copy(dst2d.at[s].at[pl.ds(c * (NCHUNK // 2), NCHUNK // 2)],
                    dst_v)
    pltpu.sync_copy(ones, ones_v)
    plsc.subcore_barrier()

    def body(i, carry):
      @pl.when(i >= NBUF)
      def _():
        pltpu.make_async_copy(ones_v, acc.at[dst_v.at[0]], ssem).wait()

      pltpu.async_copy(ones_v, acc.at[dst_v.at[i]], ssem, add=True)
      return carry

    lax.fori_loop(0, NCHUNK // 2, body, 0)

    def drain(i, carry):
      pltpu.make_async_copy(ones_v, acc.at[dst_v.at[0]], ssem).wait()
      return carry

    lax.fori_loop(0, NBUF, drain, 0)
    plsc.subcore_barrier()
    pltpu.sync_copy(acc.at[pl.ds(s * STRIPE, STRIPE)],
                    out.at[c].at[pl.ds(s * STRIPE, STRIPE)])

  return count


R = 2528
GRID = NPAD // R
TROWS = NPAD // 4
RQ = R // 4


def _pack(a):
  return jnp.concatenate([a[i * RQ:(i + 1) * RQ] for i in range(4)], axis=1)


def _unpack(q):
  return jnp.concatenate([q[:, i * HH:(i + 1) * HH] for i in range(4)],
                         axis=0)


def _perm(r):
  b = r % R
  return (r // R) * R + 4 * (b % RQ) + b // RQ


def _ln_relu(x, g, b):
  mu = jnp.mean(x, axis=-1, keepdims=True)
  var = jnp.mean((x - mu) * (x - mu), axis=-1, keepdims=True)
  return jnp.maximum((x - mu) * lax.rsqrt(var + 1e-5) * g + b, 0.0)


def _enc_body(x_ref, w_ref, b_ref, g_ref, be_ref, x0_ref, x1_ref, alo_ref,
              ahi_ref):
  h = jnp.dot(x_ref[...], w_ref[...], preferred_element_type=jnp.float32)
  h = h + b_ref[...]
  x0_ref[...] = h[:, :HG]
  x1 = h[:, HG:]
  x1_ref[...] = x1
  a = _ln_relu(x1, g_ref[...], be_ref[...])
  alo_ref[...] = _pack(a[:, :HH])
  ahi_ref[...] = _pack(a[:, HH:])


_full = lambda shape: pl.BlockSpec(shape, lambda i: (0,) * len(shape))
_rows = lambda shape: pl.BlockSpec(shape, lambda i: (i,) + (0,) * (len(shape) - 1))

_enc = pl.pallas_call(
    _enc_body,
    grid=(GRID,),
    in_specs=[_rows((R, D_IN)), _full((D_IN, HID)), _full((1, HID)),
              _full((1, HG)), _full((1, HG))],
    out_specs=[_rows((R, HG)), _rows((R, HG)), _rows((RQ, 128)),
               _rows((RQ, 128))],
    out_shape=[jax.ShapeDtypeStruct((NPAD, HG), jnp.float32)] * 2 +
              [jax.ShapeDtypeStruct((TROWS, 128), jnp.float32)] * 2,
)


_acore = lambda k: pl.BlockSpec((1, RQ, 128), lambda i, _k=k: (_k, i, 0))


def _pre_z_body(y_ref, olo_ref, ohi_ref, wr_ref, bl_ref, z_ref):
  o = jnp.concatenate([_unpack(olo_ref[...]), _unpack(ohi_ref[...])],
                      axis=-1)
  z_ref[...] = (y_ref[...] + bl_ref[...]
                + jnp.dot(o, wr_ref[...],
                          preferred_element_type=jnp.float32))


_pre_z = pl.pallas_call(
    _pre_z_body,
    grid=(GRID,),
    in_specs=[_rows((R, HG)), _rows((RQ, 128)), _rows((RQ, 128)),
              _full((HG, HG)), _full((1, HG))],
    out_specs=_rows((R, HG)),
    out_shape=jax.ShapeDtypeStruct((NPAD, HG), jnp.float32),
)


def _mean(alo, ahi, c0, c1):
  cnt = jnp.maximum(c0[:, :1] + c1[:, :1], 1.0)
  return jnp.concatenate([_unpack(alo), _unpack(ahi)], axis=-1) * (1.0 / cnt)


def _post_pre_body(z_ref, alo_ref, ahi_ref, c0_ref, c1_ref, wl_ref, g_ref,
                   b_ref, yo_ref, olo2_ref, ohi2_ref):
  mean = _mean(alo_ref[0], ahi_ref[0], c0_ref[...], c1_ref[...])
  y = z_ref[...] + jnp.dot(mean, wl_ref[...],
                           preferred_element_type=jnp.float32)
  yo_ref[...] = y
  o2 = _ln_relu(y, g_ref[...], b_ref[...])
  olo2_ref[...] = _pack(o2[:, :HH])
  ohi2_ref[...] = _pack(o2[:, HH:])


_post_pre = pl.pallas_call(
    _post_pre_body,
    grid=(GRID,),
    in_specs=[_rows((R, HG)), _acore(0), _acore(1)] +
             [_rows((R, CD))] * 2 +
             [_full((HG, HG)), _full((1, HG)), _full((1, HG))],
    out_specs=[_rows((R, HG)), _rows((RQ, 128)), _rows((RQ, 128))],
    out_shape=[jax.ShapeDtypeStruct((NPAD, HG), jnp.float32)] +
              [jax.ShapeDtypeStruct((TROWS, 128), jnp.float32)] * 2,
)


def _final_body(z_ref, alo_ref, ahi_ref, c0_ref, c1_ref, wl_ref, x0_ref,
                lg_ref, lb_ref, pw_ref, pb_ref, out_ref):
  mean = _mean(alo_ref[0], ahi_ref[0], c0_ref[...], c1_ref[...])
  y1 = z_ref[...] + jnp.dot(mean, wl_ref[...],
                            preferred_element_type=jnp.float32)
  h = jnp.concatenate([x0_ref[...], y1], axis=-1)
  hn = _ln_relu(h, lg_ref[...], lb_ref[...])
  logits = jnp.dot(hn, pw_ref[...], preferred_element_type=jnp.float32)
  logits = logits + pb_ref[...]
  m = jnp.max(logits, axis=-1, keepdims=True)
  e = logits - m
  out_ref[...] = e - jnp.log(jnp.sum(jnp.exp(e), axis=-1, keepdims=True))


_final = pl.pallas_call(
    _final_body,
    grid=(GRID,),
    in_specs=[_rows((R, HG)), _acore(0), _acore(1)] +
             [_rows((R, CD))] * 2 +
             [_full((HG, HG)), _rows((R, HG)), _full((1, HID)),
              _full((1, HID)), _full((HID, OUT)), _full((1, OUT))],
    out_specs=_rows((R, OUT)),
    out_shape=jax.ShapeDtypeStruct((N, OUT), jnp.float32),
)


def kernel(x, edge_index, enc_W, enc_b, norm_gamma, norm_beta, linl_W, linl_b,
           linr_W, last_gamma, last_beta, pred_W, pred_b):
  src = edge_index[0]
  dst = edge_index[1]
  pad = EPAD - E
  srcf = jnp.concatenate([src, jnp.zeros((pad,), jnp.int32)])
  dstf = jnp.concatenate([dst, jnp.full((pad,), N, jnp.int32)])
  src2d = _perm(srcf).reshape(NTILE, NCHUNK, CH)
  dst2d = _perm(dstf).reshape(NTILE, NCHUNK, CH)
  dst2d_nat = dstf.reshape(NTILE, NCHUNK, CH)
  zeros16 = jnp.zeros((NPAD, CD), jnp.float32)
  ones16 = jnp.ones((CH, CD), jnp.float32)

  cntp = _make_sc_count()(dst2d_nat, ones16, zeros16)
  c0 = cntp[0]
  c1 = cntp[1]

  r1 = lambda v: v.reshape(1, -1)
  x0, x1, olo, ohi = _enc(x, enc_W, r1(enc_b), r1(norm_gamma[0, 0]),
                          r1(norm_beta[0, 0]))
  res = [x0, x1]
  for l in range(LAYERS):
    for g in range(2):
      aggp = _make_sc_agg()(olo.reshape(NPAD, HH), ohi.reshape(NPAD, HH),
                            src2d, dst2d)
      aggp = aggp.reshape(NCORE, TROWS, 128)
      z = _pre_z(res[g], olo, ohi, linr_W[l, g], r1(linl_b[l, g]))
      if (l, g) == (LAYERS - 1, 1):
        return _final(z, aggp, aggp, c0, c1, linl_W[l, g], res[0],
                      r1(last_gamma), r1(last_beta), pred_W, r1(pred_b))
      nl, ng = (l, 1) if g == 0 else (l + 1, 0)
      y, olo, ohi = _post_pre(z, aggp, aggp, c0, c1, linl_W[l, g],
                              r1(norm_gamma[nl, ng]), r1(norm_beta[nl, ng]))
      res[g] = y


# speedup vs baseline: 1.7066x; 1.7066x over previous
import functools

import jax
import jax.numpy as jnp
from jax import lax
from jax.experimental import pallas as pl
from jax.experimental.pallas import tpu as pltpu
from jax.experimental.pallas import tpu_sc as plsc

N = 10000
E = 320000
D_IN = 128
HID = 128
HG = 64
HH = 32
OUT = 47
LAYERS = 4

NTILE = 16
NCORE = 2
NPAD = 10112
STRIPE = NPAD // NTILE
CH = 256
NCHUNK = 80
EPT = CH * NCHUNK
EPAD = NTILE * EPT
NBUF = 5
LEAD = 3
SPLIT = 60
CD = 16


@functools.cache
def _sc_mesh():
  return plsc.VectorSubcoreMesh(
      core_axis_name="c", subcore_axis_name="s", num_cores=NCORE,
      num_subcores=NTILE)


@functools.cache
def _make_sc_agg():
  @functools.partial(
      pl.kernel,
      out_type=pltpu.HBM((NCORE, NPAD, HH), jnp.float32),
      mesh=_sc_mesh(),
      scratch_types=[
          pltpu.VMEM((NCHUNK, CH), jnp.int32),
          pltpu.VMEM((NCHUNK, CH), jnp.int32),
          [pltpu.VMEM((CH, HH), jnp.float32)] * NBUF,
          [pltpu.SemaphoreType.DMA] * NBUF,
          [pltpu.SemaphoreType.DMA] * NBUF,
          pltpu.VMEM_SHARED((NPAD, HH), jnp.float32),
          pltpu.VMEM_SHARED((NPAD, HH), jnp.float32),
      ],
      compiler_params=pltpu.CompilerParams(use_tc_tiling_on_sc=False),
  )
  def agg(tab_lo, tab_hi, src2d, dst2d, out, src_v, dst_v, rows, gsems,
          ssems, acc, tab_s):
    c = lax.axis_index("c")
    s = lax.axis_index("s")
    zv = jnp.zeros((16,), jnp.float32)

    def zfill(i, carry):
      r = i // (HH // 16)
      k = i % (HH // 16)
      rows[0][r, pl.ds(k * 16, 16)] = zv
      return carry

    lax.fori_loop(0, CH * HH // 16, zfill, 0)
    nz = STRIPE // CH
    for z in range(nz):
      pltpu.sync_copy(rows[0], acc.at[pl.ds(s * STRIPE + z * CH, CH)])
    rem = STRIPE - nz * CH
    if rem:
      pltpu.sync_copy(
          rows[0].at[pl.ds(0, rem)],
          acc.at[pl.ds(s * STRIPE + nz * CH, rem)])

    @pl.when(c == 0)
    def _():
      pltpu.sync_copy(tab_lo.at[pl.ds(s * STRIPE, STRIPE)],
                      tab_s.at[pl.ds(s * STRIPE, STRIPE)])

    @pl.when(c == 1)
    def _():
      pltpu.sync_copy(tab_hi.at[pl.ds(s * STRIPE, STRIPE)],
                      tab_s.at[pl.ds(s * STRIPE, STRIPE)])

    pltpu.sync_copy(src2d.at[s], src_v)
    pltpu.sync_copy(dst2d.at[s], dst_v)
    plsc.subcore_barrier()

    def gather_start(i, buf, sem):
      pltpu.async_copy(tab_s.at[src_v.at[i]], buf, sem)

    for i in range(LEAD):
      gather_start(jnp.int32(i), rows[i], gsems[i])

    def body(g, carry):
      for b in range(NBUF):
        i = g * NBUF + b
        pltpu.make_async_copy(tab_s.at[src_v.at[i]], rows[b],
                              gsems[b]).wait()
        pltpu.async_copy(rows[b], acc.at[dst_v.at[i]], ssems[b], add=True)
        j = i + LEAD
        bj = (b + LEAD) % NBUF

        @pl.when(j < NCHUNK)
        def _():
          @pl.when(j >= NBUF)
          def _():
            pltpu.make_async_copy(rows[bj], acc.at[dst_v.at[0]],
                                  ssems[bj]).wait()
          gather_start(j, rows[bj], gsems[bj])

      return carry

    lax.fori_loop(0, NCHUNK // NBUF, body, 0)
    for b in range(NBUF):
      pltpu.make_async_copy(rows[b], acc.at[dst_v.at[0]], ssems[b]).wait()
    plsc.subcore_barrier()
    pltpu.sync_copy(acc.at[pl.ds(s * STRIPE, STRIPE)],
                    out.at[c].at[pl.ds(s * STRIPE, STRIPE)])

  return agg


@functools.cache
def _make_sc_count():

  @functools.partial(
      pl.kernel,
      out_type=pltpu.HBM((NCORE, NPAD, CD), jnp.float32),
      mesh=_sc_mesh(),
      scratch_types=[
          pltpu.VMEM((NCHUNK // 2, CH), jnp.int32),
          pltpu.VMEM((CH, CD), jnp.float32),
          pltpu.VMEM_SHARED((NPAD, CD), jnp.float32),
          pltpu.SemaphoreType.DMA,
      ],
      compiler_params=pltpu.CompilerParams(use_tc_tiling_on_sc=False),
  )
  def count(dst2d, ones, zeros, out, dst_v, ones_v, acc, ssem):
    c = lax.axis_index("c")
    s = lax.axis_index("s")
    pltpu.sync_copy(zeros.at[pl.ds(s * STRIPE, STRIPE)],
                    acc.at[pl.ds(s * STRIPE, STRIPE)])
    pltpu.sync_copy(dst2d.at[s].at[pl.ds(c * (NCHUNK // 2), NCHUNK // 2)],
                    dst_v)
    pltpu.sync_copy(ones, ones_v)
    plsc.subcore_barrier()

    def body(i, carry):
      @pl.when(i >= NBUF)
      def _():
        pltpu.make_async_copy(ones_v, acc.at[dst_v.at[0]], ssem).wait()

      pltpu.async_copy(ones_v, acc.at[dst_v.at[i]], ssem, add=True)
      return carry

    lax.fori_loop(0, NCHUNK // 2, body, 0)

    def drain(i, carry):
      pltpu.make_async_copy(ones_v, acc.at[dst_v.at[0]], ssem).wait()
      return carry

    lax.fori_loop(0, NBUF, drain, 0)
    plsc.subcore_barrier()
    pltpu.sync_copy(acc.at[pl.ds(s * STRIPE, STRIPE)],
                    out.at[c].at[pl.ds(s * STRIPE, STRIPE)])

  return count


R = 2528
GRID = NPAD // R
TROWS = NPAD // 4
RQ = R // 4


def _pack(a):
  return jnp.concatenate([a[i * RQ:(i + 1) * RQ] for i in range(4)], axis=1)


def _unpack(q):
  return jnp.concatenate([q[:, i * HH:(i + 1) * HH] for i in range(4)],
                         axis=0)


def _perm(r):
  b = r % R
  return (r // R) * R + 4 * (b % RQ) + b // RQ


def _ln_relu(x, g, b):
  mu = jnp.mean(x, axis=-1, keepdims=True)
  var = jnp.mean((x - mu) * (x - mu), axis=-1, keepdims=True)
  return jnp.maximum((x - mu) * lax.rsqrt(var + 1e-5) * g + b, 0.0)


def _enc_body(x_ref, w_ref, b_ref, g_ref, be_ref, x0_ref, x1_ref, alo_ref,
              ahi_ref):
  h = jnp.dot(x_ref[...], w_ref[...], preferred_element_type=jnp.float32)
  h = h + b_ref[...]
  x0_ref[...] = h[:, :HG]
  x1 = h[:, HG:]
  x1_ref[...] = x1
  a = _ln_relu(x1, g_ref[...], be_ref[...])
  alo_ref[...] = _pack(a[:, :HH])
  ahi_ref[...] = _pack(a[:, HH:])


_full = lambda shape: pl.BlockSpec(shape, lambda i: (0,) * len(shape))
_rows = lambda shape: pl.BlockSpec(shape, lambda i: (i,) + (0,) * (len(shape) - 1))

_enc = pl.pallas_call(
    _enc_body,
    grid=(GRID,),
    in_specs=[_rows((R, D_IN)), _full((D_IN, HID)), _full((1, HID)),
              _full((1, HG)), _full((1, HG))],
    out_specs=[_rows((R, HG)), _rows((R, HG)), _rows((RQ, 128)),
               _rows((RQ, 128))],
    out_shape=[jax.ShapeDtypeStruct((NPAD, HG), jnp.float32)] * 2 +
              [jax.ShapeDtypeStruct((TROWS, 128), jnp.float32)] * 2,
)


_acore = lambda k: pl.BlockSpec((1, RQ, 128), lambda i, _k=k: (_k, i, 0))


def _pre_z_body(y_ref, olo_ref, ohi_ref, wr_ref, bl_ref, z_ref):
  o = jnp.concatenate([_unpack(olo_ref[...]), _unpack(ohi_ref[...])],
                      axis=-1)
  z_ref[...] = (y_ref[...] + bl_ref[...]
                + jnp.dot(o, wr_ref[...],
                          preferred_element_type=jnp.float32))


_pre_z = pl.pallas_call(
    _pre_z_body,
    grid=(GRID,),
    in_specs=[_rows((R, HG)), _rows((RQ, 128)), _rows((RQ, 128)),
              _full((HG, HG)), _full((1, HG))],
    out_specs=_rows((R, HG)),
    out_shape=jax.ShapeDtypeStruct((NPAD, HG), jnp.float32),
)


def _mean(alo, ahi, c0, c1):
  cnt = jnp.maximum(c0[:, :1] + c1[:, :1], 1.0)
  return jnp.concatenate([_unpack(alo), _unpack(ahi)], axis=-1) * (1.0 / cnt)


def _post_pre_body(z_ref, alo_ref, ahi_ref, c0_ref, c1_ref, wl_ref, g_ref,
                   b_ref, yo_ref, olo2_ref, ohi2_ref):
  mean = _mean(alo_ref[0], ahi_ref[0], c0_ref[...], c1_ref[...])
  y = z_ref[...] + jnp.dot(mean, wl_ref[...],
                           preferred_element_type=jnp.float32)
  yo_ref[...] = y
  o2 = _ln_relu(y, g_ref[...], b_ref[...])
  olo2_ref[...] = _pack(o2[:, :HH])
  ohi2_ref[...] = _pack(o2[:, HH:])


_post_pre = pl.pallas_call(
    _post_pre_body,
    grid=(GRID,),
    in_specs=[_rows((R, HG)), _acore(0), _acore(1)] +
             [_rows((R, CD))] * 2 +
             [_full((HG, HG)), _full((1, HG)), _full((1, HG))],
    out_specs=[_rows((R, HG)), _rows((RQ, 128)), _rows((RQ, 128))],
    out_shape=[jax.ShapeDtypeStruct((NPAD, HG), jnp.float32)] +
              [jax.ShapeDtypeStruct((TROWS, 128), jnp.float32)] * 2,
)


def _final_body(z_ref, alo_ref, ahi_ref, c0_ref, c1_ref, wl_ref, x0_ref,
                lg_ref, lb_ref, pw_ref, pb_ref, out_ref):
  mean = _mean(alo_ref[0], ahi_ref[0], c0_ref[...], c1_ref[...])
  y1 = z_ref[...] + jnp.dot(mean, wl_ref[...],
                            preferred_element_type=jnp.float32)
  h = jnp.concatenate([x0_ref[...], y1], axis=-1)
  hn = _ln_relu(h, lg_ref[...], lb_ref[...])
  logits = jnp.dot(hn, pw_ref[...], preferred_element_type=jnp.float32)
  logits = logits + pb_ref[...]
  m = jnp.max(logits, axis=-1, keepdims=True)
  e = logits - m
  out_ref[...] = e - jnp.log(jnp.sum(jnp.exp(e), axis=-1, keepdims=True))


_final = pl.pallas_call(
    _final_body,
    grid=(GRID,),
    in_specs=[_rows((R, HG)), _acore(0), _acore(1)] +
             [_rows((R, CD))] * 2 +
             [_full((HG, HG)), _rows((R, HG)), _full((1, HID)),
              _full((1, HID)), _full((HID, OUT)), _full((1, OUT))],
    out_specs=_rows((R, OUT)),
    out_shape=jax.ShapeDtypeStruct((N, OUT), jnp.float32),
)


def kernel(x, edge_index, enc_W, enc_b, norm_gamma, norm_beta, linl_W, linl_b,
           linr_W, last_gamma, last_beta, pred_W, pred_b):
  src = edge_index[0]
  dst = edge_index[1]
  pad = EPAD - E
  srcf = jnp.concatenate([src, jnp.zeros((pad,), jnp.int32)])
  dstf = jnp.concatenate([dst, jnp.full((pad,), N, jnp.int32)])
  src2d = _perm(srcf).reshape(NTILE, NCHUNK, CH)
  dst2d = _perm(dstf).reshape(NTILE, NCHUNK, CH)
  dst2d_nat = dstf.reshape(NTILE, NCHUNK, CH)
  zeros16 = jnp.zeros((NPAD, CD), jnp.float32)
  ones16 = jnp.ones((CH, CD), jnp.float32)

  cntp = _make_sc_count()(dst2d_nat, ones16, zeros16)
  c0 = cntp[0]
  c1 = cntp[1]

  r1 = lambda v: v.reshape(1, -1)
  x0, x1, olo, ohi = _enc(x, enc_W, r1(enc_b), r1(norm_gamma[0, 0]),
                          r1(norm_beta[0, 0]))
  res = [x0, x1]
  for l in range(LAYERS):
    for g in range(2):
      aggp = _make_sc_agg()(olo.reshape(NPAD, HH), ohi.reshape(NPAD, HH),
                            src2d, dst2d)
      aggp = aggp.reshape(NCORE, TROWS, 128)
      z = _pre_z(res[g], olo, ohi, linr_W[l, g], r1(linl_b[l, g]))
      if (l, g) == (LAYERS - 1, 1):
        return _final(z, aggp, aggp, c0, c1, linl_W[l, g], res[0],
                      r1(last_gamma), r1(last_beta), pred_W, r1(pred_b))
      nl, ng = (l, 1) if g == 0 else (l + 1, 0)
      y, olo, ohi = _post_pre(z, aggp, aggp, c0, c1, linl_W[l, g],
                              r1(norm_gamma[nl, ng]), r1(norm_beta[nl, ng]))
      res[g] = y
